# baseline (speedup 1.0000x reference)
import jax
import jax.numpy as jnp
from jax import lax
from jax.experimental import pallas as pl
from jax.experimental.pallas import tpu as pltpu
from jax.experimental.pallas import tpu_sc as plsc

_B = 16384
_L = 50
_D = 128
_TOT = _B * _L
_NC = 2
_NS = 16
_NW = _NC * _NS
_PER_W = _TOT // _NW
_CHUNK = 128
_NCH = _PER_W // _CHUNK

_MM_BLK = 5000


def _proj_body(x_ref, w_ref, o_ref):
    o_ref[...] = lax.dot_general(
        x_ref[...], w_ref[...],
        (((1,), (1,)), ((), ())),
        preferred_element_type=jnp.float32,
    )


def _project_table(emb, W):
    m = emb.shape[0]
    grid = m // _MM_BLK
    return pl.pallas_call(
        _proj_body,
        grid=(grid,),
        in_specs=[
            pl.BlockSpec((_MM_BLK, _D), lambda i: (i, 0)),
            pl.BlockSpec((_D, _D), lambda i: (0, 0)),
        ],
        out_specs=pl.BlockSpec((_MM_BLK, _D), lambda i: (i, 0)),
        out_shape=jax.ShapeDtypeStruct((m, _D), jnp.float32),
    )(emb, W)


_PER_W = _TOT // _NW
_IROWS = 1
_CHUNK = _IROWS * 128
_NIR = _PER_W // 128
_NCH = _PER_W // _CHUNK


_NBUF = 4
_NOUT = _NCH // _NBUF


def _gather_body(tab_hbm, idx_hbm, out_hbm, idx_v,
                 r0, r1, r2, r3, g0, g1, g2, g3, w0, w1, w2, w3):
    rows = [r0, r1, r2, r3]
    gsem = [g0, g1, g2, g3]
    wsem = [w0, w1, w2, w3]
    wid = lax.axis_index("s") * _NC + lax.axis_index("c")
    pltpu.sync_copy(idx_hbm.at[pl.ds(wid * _NIR, _NIR)], idx_v)
    out_base = wid * _PER_W

    fire = lambda j, b: pltpu.async_copy(
        tab_hbm.at[idx_v.at[j]], rows[b], gsem[b])
    drain = lambda b: pltpu.make_async_copy(
        tab_hbm.at[idx_v.at[0]], rows[b], gsem[b]).wait()
    store = lambda j, b: pltpu.async_copy(
        rows[b], out_hbm.at[pl.ds(out_base + j * _CHUNK, _CHUNK)], wsem[b])
    drain_store = lambda b: pltpu.make_async_copy(
        rows[b], out_hbm.at[pl.ds(0, _CHUNK)], wsem[b]).wait()

    for c in range(_NBUF - 1):
        fire(c, c)

    def outer(jo, carry):
        for k in range(_NBUF):
            c = _NBUF * jo + k
            drain(k)
            store(c, k)
            nb = (k + _NBUF - 1) % _NBUF

            @pl.when(c + _NBUF - 1 < _NCH)
            def _():
                @pl.when(c >= 1)
                def _():
                    drain_store(nb)
                fire(c + _NBUF - 1, nb)

        return carry

    lax.fori_loop(0, _NOUT, outer, 0)
    for k in range(_NBUF):
        drain_store(k)


_gather = pl.kernel(
    _gather_body,
    out_type=jax.ShapeDtypeStruct((_TOT, _D), jnp.float32),
    mesh=plsc.VectorSubcoreMesh(
        core_axis_name="c", subcore_axis_name="s",
        num_cores=_NC, num_subcores=_NS,
    ),
    scratch_types=[
        pltpu.VMEM((_NIR, 128), jnp.int32),
        pltpu.VMEM((_CHUNK, _D), jnp.float32),
        pltpu.VMEM((_CHUNK, _D), jnp.float32),
        pltpu.VMEM((_CHUNK, _D), jnp.float32),
        pltpu.VMEM((_CHUNK, _D), jnp.float32),
        pltpu.SemaphoreType.DMA,
        pltpu.SemaphoreType.DMA,
        pltpu.SemaphoreType.DMA,
        pltpu.SemaphoreType.DMA,
        pltpu.SemaphoreType.DMA,
        pltpu.SemaphoreType.DMA,
        pltpu.SemaphoreType.DMA,
        pltpu.SemaphoreType.DMA,
    ],
)


@jax.jit
def kernel(idx, emb, W):
    proj = _project_table(emb, W)
    idx_t = idx.astype(jnp.int32).T.reshape(_NW * _NIR, 128)
    flat = _gather(proj, idx_t)
    return flat.reshape(_L, _B, _D).transpose(1, 0, 2)


# speedup vs baseline: 1.0019x; 1.0019x over previous
import jax
import jax.numpy as jnp
from jax import lax
from jax.experimental import pallas as pl
from jax.experimental.pallas import tpu as pltpu
from jax.experimental.pallas import tpu_sc as plsc

_B = 16384
_L = 50
_D = 128
_TOT = _B * _L
_NC = 2
_NS = 16
_NW = _NC * _NS
_PER_W = _TOT // _NW
_CHUNK = 128
_NCH = _PER_W // _CHUNK

_MM_BLK = 5000


def _proj_body(x_ref, w_ref, o_ref):
    o_ref[...] = lax.dot_general(
        x_ref[...], w_ref[...],
        (((1,), (1,)), ((), ())),
        preferred_element_type=jnp.float32,
    )


def _project_table(emb, W):
    m = emb.shape[0]
    grid = m // _MM_BLK
    return pl.pallas_call(
        _proj_body,
        grid=(grid,),
        in_specs=[
            pl.BlockSpec((_MM_BLK, _D), lambda i: (i, 0)),
            pl.BlockSpec((_D, _D), lambda i: (0, 0)),
        ],
        out_specs=pl.BlockSpec((_MM_BLK, _D), lambda i: (i, 0)),
        out_shape=jax.ShapeDtypeStruct((m, _D), jnp.float32),
    )(emb, W)


_PER_W = _TOT // _NW
_IROWS = 1
_CHUNK = _IROWS * 128
_NIR = _PER_W // 128
_NCH = _PER_W // _CHUNK


_SPC = 2
_OCH = _SPC * _CHUNK
_NST = _NCH // _SPC


def _gather_body(tab_hbm, idx_hbm, out_hbm, idx_v, rows_a, rows_b,
                 gsem_a, gsem_b, wsem_a, wsem_b):
    wid = lax.axis_index("s") * _NC + lax.axis_index("c")
    pltpu.sync_copy(idx_hbm.at[pl.ds(wid * _NIR, _NIR)], idx_v)
    out_base = wid * _PER_W

    def fire_pair(j, rows, gsem):
        for k in range(_SPC):
            pltpu.async_copy(
                tab_hbm.at[idx_v.at[j * _SPC + k]],
                rows.at[pl.ds(k * _CHUNK, _CHUNK)], gsem)

    def drain_pair(rows, gsem):
        for k in range(_SPC):
            pltpu.make_async_copy(
                tab_hbm.at[idx_v.at[0]],
                rows.at[pl.ds(k * _CHUNK, _CHUNK)], gsem).wait()

    store = lambda j, rows, wsem: pltpu.async_copy(
        rows, out_hbm.at[pl.ds(out_base + j * _OCH, _OCH)], wsem)
    drain_store = lambda rows, wsem: pltpu.make_async_copy(
        rows, out_hbm.at[pl.ds(0, _OCH)], wsem).wait()

    fire_pair(0, rows_a, gsem_a)

    def step(j, carry):
        even = (j % 2) == 0

        @pl.when(jnp.logical_and(j + 1 < _NST, even))
        def _():
            @pl.when(j >= 1)
            def _():
                drain_store(rows_b, wsem_b)
            fire_pair(j + 1, rows_b, gsem_b)

        @pl.when(jnp.logical_and(j + 1 < _NST, jnp.logical_not(even)))
        def _():
            drain_store(rows_a, wsem_a)
            fire_pair(j + 1, rows_a, gsem_a)

        @pl.when(even)
        def _():
            drain_pair(rows_a, gsem_a)
            store(j, rows_a, wsem_a)

        @pl.when(jnp.logical_not(even))
        def _():
            drain_pair(rows_b, gsem_b)
            store(j, rows_b, wsem_b)

        return carry

    lax.fori_loop(0, _NST, step, 0)
    drain_store(rows_a, wsem_a)
    drain_store(rows_b, wsem_b)


_gather = pl.kernel(
    _gather_body,
    out_type=jax.ShapeDtypeStruct((_TOT, _D), jnp.float32),
    mesh=plsc.VectorSubcoreMesh(
        core_axis_name="c", subcore_axis_name="s",
        num_cores=_NC, num_subcores=_NS,
    ),
    scratch_types=[
        pltpu.VMEM((_NIR, 128), jnp.int32),
        pltpu.VMEM((_OCH, _D), jnp.float32),
        pltpu.VMEM((_OCH, _D), jnp.float32),
        pltpu.SemaphoreType.DMA,
        pltpu.SemaphoreType.DMA,
        pltpu.SemaphoreType.DMA,
        pltpu.SemaphoreType.DMA,
    ],
)


@jax.jit
def kernel(idx, emb, W):
    proj = _project_table(emb, W)
    idx_t = idx.astype(jnp.int32).T.reshape(_NW * _NIR, 128)
    flat = _gather(proj, idx_t)
    return flat.reshape(_L, _B, _D).transpose(1, 0, 2)


# speedup vs baseline: 1.0172x; 1.0153x over previous
import jax
import jax.numpy as jnp
from jax import lax
from jax.experimental import pallas as pl
from jax.experimental.pallas import tpu as pltpu
from jax.experimental.pallas import tpu_sc as plsc

_B = 16384
_L = 50
_D = 128
_TOT = _B * _L
_NC = 2
_NS = 16
_NW = _NC * _NS
_PER_W = _TOT // _NW
_CHUNK = 128
_NCH = _PER_W // _CHUNK

_MM_BLK = 10000


def _proj_body(x_ref, w_ref, o_ref):
    o_ref[...] = lax.dot_general(
        x_ref[...], w_ref[...],
        (((1,), (1,)), ((), ())),
        preferred_element_type=jnp.float32,
    )


def _project_table(emb, W):
    m = emb.shape[0]
    grid = m // _MM_BLK
    return pl.pallas_call(
        _proj_body,
        grid=(grid,),
        in_specs=[
            pl.BlockSpec((_MM_BLK, _D), lambda i: (i, 0)),
            pl.BlockSpec((_D, _D), lambda i: (0, 0)),
        ],
        out_specs=pl.BlockSpec((_MM_BLK, _D), lambda i: (i, 0)),
        out_shape=jax.ShapeDtypeStruct((m, _D), jnp.float32),
    )(emb, W)


_PER_W = _TOT // _NW
_IROWS = 1
_CHUNK = _IROWS * 128
_NIR = _PER_W // 128
_NCH = _PER_W // _CHUNK


_SPC = 2
_OCH = _SPC * _CHUNK
_NST = _NCH // _SPC


def _gather_body(tab_hbm, idx_hbm, out_hbm, idx_v, rows_a, rows_b,
                 gsem_a, gsem_b, wsem_a, wsem_b):
    wid = lax.axis_index("s") * _NC + lax.axis_index("c")
    pltpu.sync_copy(idx_hbm.at[pl.ds(wid * _NIR, _NIR)], idx_v)
    out_base = wid * _PER_W

    def fire_pair(j, rows, gsem):
        for k in range(_SPC):
            pltpu.async_copy(
                tab_hbm.at[idx_v.at[j * _SPC + k]],
                rows.at[pl.ds(k * _CHUNK, _CHUNK)], gsem)

    def drain_pair(rows, gsem):
        for k in range(_SPC):
            pltpu.make_async_copy(
                tab_hbm.at[idx_v.at[0]],
                rows.at[pl.ds(k * _CHUNK, _CHUNK)], gsem).wait()

    store = lambda j, rows, wsem: pltpu.async_copy(
        rows, out_hbm.at[pl.ds(out_base + j * _OCH, _OCH)], wsem)
    drain_store = lambda rows, wsem: pltpu.make_async_copy(
        rows, out_hbm.at[pl.ds(0, _OCH)], wsem).wait()

    fire_pair(0, rows_a, gsem_a)

    def step(j, carry):
        even = (j % 2) == 0

        @pl.when(jnp.logical_and(j + 1 < _NST, even))
        def _():
            @pl.when(j >= 1)
            def _():
                drain_store(rows_b, wsem_b)
            fire_pair(j + 1, rows_b, gsem_b)

        @pl.when(jnp.logical_and(j + 1 < _NST, jnp.logical_not(even)))
        def _():
            drain_store(rows_a, wsem_a)
            fire_pair(j + 1, rows_a, gsem_a)

        @pl.when(even)
        def _():
            drain_pair(rows_a, gsem_a)
            store(j, rows_a, wsem_a)

        @pl.when(jnp.logical_not(even))
        def _():
            drain_pair(rows_b, gsem_b)
            store(j, rows_b, wsem_b)

        return carry

    lax.fori_loop(0, _NST, step, 0)
    drain_store(rows_a, wsem_a)
    drain_store(rows_b, wsem_b)


_gather = pl.kernel(
    _gather_body,
    out_type=jax.ShapeDtypeStruct((_TOT, _D), jnp.float32),
    mesh=plsc.VectorSubcoreMesh(
        core_axis_name="c", subcore_axis_name="s",
        num_cores=_NC, num_subcores=_NS,
    ),
    scratch_types=[
        pltpu.VMEM((_NIR, 128), jnp.int32),
        pltpu.VMEM((_OCH, _D), jnp.float32),
        pltpu.VMEM((_OCH, _D), jnp.float32),
        pltpu.SemaphoreType.DMA,
        pltpu.SemaphoreType.DMA,
        pltpu.SemaphoreType.DMA,
        pltpu.SemaphoreType.DMA,
    ],
)


@jax.jit
def kernel(idx, emb, W):
    proj = _project_table(emb, W)
    idx_t = idx.astype(jnp.int32).T.reshape(_NW * _NIR, 128)
    flat = _gather(proj, idx_t)
    return flat.reshape(_L, _B, _D).transpose(1, 0, 2)


# speedup vs baseline: 1.0219x; 1.0046x over previous
import jax
import jax.numpy as jnp
from jax import lax
from jax.experimental import pallas as pl
from jax.experimental.pallas import tpu as pltpu
from jax.experimental.pallas import tpu_sc as plsc

_B = 16384
_L = 50
_D = 128
_TOT = _B * _L
_NC = 2
_NS = 16
_NW = _NC * _NS
_PER_W = _TOT // _NW
_CHUNK = 128
_NCH = _PER_W // _CHUNK

_MM_BLK = 20000


def _proj_body(x_ref, w_ref, o_ref):
    o_ref[...] = lax.dot_general(
        x_ref[...], w_ref[...],
        (((1,), (1,)), ((), ())),
        preferred_element_type=jnp.float32,
    )


def _project_table(emb, W):
    m = emb.shape[0]
    grid = m // _MM_BLK
    return pl.pallas_call(
        _proj_body,
        grid=(grid,),
        in_specs=[
            pl.BlockSpec((_MM_BLK, _D), lambda i: (i, 0)),
            pl.BlockSpec((_D, _D), lambda i: (0, 0)),
        ],
        out_specs=pl.BlockSpec((_MM_BLK, _D), lambda i: (i, 0)),
        out_shape=jax.ShapeDtypeStruct((m, _D), jnp.float32),
    )(emb, W)


_PER_W = _TOT // _NW
_IROWS = 1
_CHUNK = _IROWS * 128
_NIR = _PER_W // 128
_NCH = _PER_W // _CHUNK


_SPC = 2
_OCH = _SPC * _CHUNK
_NST = _NCH // _SPC


def _gather_body(tab_hbm, idx_hbm, out_hbm, idx_v, rows_a, rows_b,
                 gsem_a, gsem_b, wsem_a, wsem_b):
    wid = lax.axis_index("s") * _NC + lax.axis_index("c")
    pltpu.sync_copy(idx_hbm.at[pl.ds(wid * _NIR, _NIR)], idx_v)
    out_base = wid * _PER_W

    def fire_pair(j, rows, gsem):
        for k in range(_SPC):
            pltpu.async_copy(
                tab_hbm.at[idx_v.at[j * _SPC + k]],
                rows.at[pl.ds(k * _CHUNK, _CHUNK)], gsem)

    def drain_pair(rows, gsem):
        for k in range(_SPC):
            pltpu.make_async_copy(
                tab_hbm.at[idx_v.at[0]],
                rows.at[pl.ds(k * _CHUNK, _CHUNK)], gsem).wait()

    store = lambda j, rows, wsem: pltpu.async_copy(
        rows, out_hbm.at[pl.ds(out_base + j * _OCH, _OCH)], wsem)
    drain_store = lambda rows, wsem: pltpu.make_async_copy(
        rows, out_hbm.at[pl.ds(0, _OCH)], wsem).wait()

    fire_pair(0, rows_a, gsem_a)

    def step(j, carry):
        even = (j % 2) == 0

        @pl.when(jnp.logical_and(j + 1 < _NST, even))
        def _():
            @pl.when(j >= 1)
            def _():
                drain_store(rows_b, wsem_b)
            fire_pair(j + 1, rows_b, gsem_b)

        @pl.when(jnp.logical_and(j + 1 < _NST, jnp.logical_not(even)))
        def _():
            drain_store(rows_a, wsem_a)
            fire_pair(j + 1, rows_a, gsem_a)

        @pl.when(even)
        def _():
            drain_pair(rows_a, gsem_a)
            store(j, rows_a, wsem_a)

        @pl.when(jnp.logical_not(even))
        def _():
            drain_pair(rows_b, gsem_b)
            store(j, rows_b, wsem_b)

        return carry

    lax.fori_loop(0, _NST, step, 0)
    drain_store(rows_a, wsem_a)
    drain_store(rows_b, wsem_b)


_gather = pl.kernel(
    _gather_body,
    out_type=jax.ShapeDtypeStruct((_TOT, _D), jnp.float32),
    mesh=plsc.VectorSubcoreMesh(
        core_axis_name="c", subcore_axis_name="s",
        num_cores=_NC, num_subcores=_NS,
    ),
    scratch_types=[
        pltpu.VMEM((_NIR, 128), jnp.int32),
        pltpu.VMEM((_OCH, _D), jnp.float32),
        pltpu.VMEM((_OCH, _D), jnp.float32),
        pltpu.SemaphoreType.DMA,
        pltpu.SemaphoreType.DMA,
        pltpu.SemaphoreType.DMA,
        pltpu.SemaphoreType.DMA,
    ],
)


@jax.jit
def kernel(idx, emb, W):
    proj = _project_table(emb, W)
    idx_t = idx.astype(jnp.int32).T.reshape(_NW * _NIR, 128)
    flat = _gather(proj, idx_t)
    return flat.reshape(_L, _B, _D).transpose(1, 0, 2)
